# baseline (speedup 1.0000x reference)
import functools

import jax
import jax.numpy as jnp
from jax import lax
from jax.experimental import pallas as pl
from jax.experimental.pallas import tpu as pltpu
from jax.experimental.pallas import tpu_sc as plsc

_NORM = 0.1481847 / 14.3996

_NC = 2
_NS = 16
_NW = _NC * _NS
_L = 16

_TBL_PAD = 128
_NB = 8


def _make_lookup(n, tbl_n):
    assert n % (_NB * _NW * _L) == 0
    per_w = n // _NW
    mesh = plsc.VectorSubcoreMesh(
        core_axis_name="c", subcore_axis_name="s",
        num_cores=_NC, num_subcores=_NS,
    )

    @functools.partial(
        pl.kernel,
        out_type=jax.ShapeDtypeStruct((n,), jnp.float32),
        mesh=mesh,
        scratch_types=[
            pltpu.VMEM((_TBL_PAD,), jnp.float32),
            [pltpu.VMEM((per_w // _NB,), jnp.int32) for _ in range(2)],
            [pltpu.VMEM((per_w // _NB,), jnp.float32) for _ in range(2)],
            [pltpu.SemaphoreType.DMA for _ in range(2)],
            [pltpu.SemaphoreType.DMA for _ in range(2)],
        ],
        compiler_params=pltpu.CompilerParams(
            needs_layout_passes=False,
            disable_bounds_checks=True,
            disable_semaphore_checks=True,
        ),
    )
    def lookup(an_hbm, tbl_hbm, out_hbm, tbl_v, idx_v, val_v, in_sem, out_sem):
        wid = lax.axis_index("s") * _NC + lax.axis_index("c")
        base = wid * per_w
        chunk = per_w // _NB

        in_cp = [None] * _NB
        out_cp = [None] * _NB
        in_cp[0] = pltpu.async_copy(
            an_hbm.at[pl.ds(base, chunk)], idx_v[0], in_sem[0]
        )

        pltpu.sync_copy(tbl_hbm, tbl_v.at[pl.ds(0, tbl_n)])
        for j in range(_TBL_PAD // _L):
            sl = pl.ds(j * _L, _L)
            tbl_v[sl] = tbl_v[sl] * _NORM

        for k in range(_NB):
            b = k % 2
            if k + 1 < _NB:
                nb = (k + 1) % 2
                in_cp[k + 1] = pltpu.async_copy(
                    an_hbm.at[pl.ds(base + (k + 1) * chunk, chunk)],
                    idx_v[nb], in_sem[nb],
                )
            in_cp[k].wait()
            if k >= 2:
                out_cp[k - 2].wait()

            @plsc.parallel_loop(0, chunk, step=_L, unroll=32)
            def _(i, ib=idx_v[b], vb=val_v[b]):
                sl = pl.ds(i, _L)
                vb[sl] = plsc.load_gather(tbl_v, [ib[sl]])

            out_cp[k] = pltpu.async_copy(
                val_v[b], out_hbm.at[pl.ds(base + k * chunk, chunk)],
                out_sem[b],
            )

        out_cp[_NB - 2].wait()
        out_cp[_NB - 1].wait()

    return lookup


def kernel(atomic_numbers, alpha_table):
    return _make_lookup(atomic_numbers.shape[0], alpha_table.shape[0])(
        atomic_numbers, alpha_table
    )


# speedup vs baseline: 1.1504x; 1.1504x over previous
import functools

import jax
import jax.numpy as jnp
from jax import lax
from jax.experimental import pallas as pl
from jax.experimental.pallas import tpu as pltpu
from jax.experimental.pallas import tpu_sc as plsc

_NORM = 0.1481847 / 14.3996

_NC = 2
_NS = 16
_NW = _NC * _NS
_L = 16

_TBL_PAD = 128
_NB = 2


def _make_lookup(n, tbl_n):
    assert n % (_NB * _NW * _L) == 0
    per_w = n // _NW
    mesh = plsc.VectorSubcoreMesh(
        core_axis_name="c", subcore_axis_name="s",
        num_cores=_NC, num_subcores=_NS,
    )

    @functools.partial(
        pl.kernel,
        out_type=jax.ShapeDtypeStruct((n,), jnp.float32),
        mesh=mesh,
        scratch_types=[
            pltpu.VMEM((_TBL_PAD,), jnp.float32),
            [pltpu.VMEM((per_w // _NB,), jnp.int32) for _ in range(2)],
            [pltpu.VMEM((per_w // _NB,), jnp.float32) for _ in range(2)],
            [pltpu.SemaphoreType.DMA for _ in range(2)],
            [pltpu.SemaphoreType.DMA for _ in range(2)],
        ],
        compiler_params=pltpu.CompilerParams(
            needs_layout_passes=False,
            disable_bounds_checks=True,
            disable_semaphore_checks=True,
        ),
    )
    def lookup(an_hbm, tbl_hbm, out_hbm, tbl_v, idx_v, val_v, in_sem, out_sem):
        wid = lax.axis_index("s") * _NC + lax.axis_index("c")
        base = wid * per_w
        chunk = per_w // _NB

        in_cp = [None] * _NB
        out_cp = [None] * _NB
        in_cp[0] = pltpu.async_copy(
            an_hbm.at[pl.ds(base, chunk)], idx_v[0], in_sem[0]
        )

        pltpu.sync_copy(tbl_hbm, tbl_v.at[pl.ds(0, tbl_n)])
        for j in range(_TBL_PAD // _L):
            sl = pl.ds(j * _L, _L)
            tbl_v[sl] = tbl_v[sl] * _NORM

        for k in range(_NB):
            b = k % 2
            if k + 1 < _NB:
                nb = (k + 1) % 2
                in_cp[k + 1] = pltpu.async_copy(
                    an_hbm.at[pl.ds(base + (k + 1) * chunk, chunk)],
                    idx_v[nb], in_sem[nb],
                )
            in_cp[k].wait()
            if k >= 2:
                out_cp[k - 2].wait()

            @plsc.parallel_loop(0, chunk, step=_L, unroll=32)
            def _(i, ib=idx_v[b], vb=val_v[b]):
                sl = pl.ds(i, _L)
                vb[sl] = plsc.load_gather(tbl_v, [ib[sl]])

            out_cp[k] = pltpu.async_copy(
                val_v[b], out_hbm.at[pl.ds(base + k * chunk, chunk)],
                out_sem[b],
            )

        out_cp[_NB - 2].wait()
        out_cp[_NB - 1].wait()

    return lookup


def kernel(atomic_numbers, alpha_table):
    return _make_lookup(atomic_numbers.shape[0], alpha_table.shape[0])(
        atomic_numbers, alpha_table
    )


# speedup vs baseline: 1.1516x; 1.0010x over previous
import functools

import jax
import jax.numpy as jnp
from jax import lax
from jax.experimental import pallas as pl
from jax.experimental.pallas import tpu as pltpu
from jax.experimental.pallas import tpu_sc as plsc

_NORM = 0.1481847 / 14.3996

_NC = 2
_NS = 16
_NW = _NC * _NS
_L = 16

_TBL_PAD = 128
_NB = 2


def _make_lookup(n, tbl_n):
    assert n % (_NB * _NW * _L) == 0
    per_w = n // _NW
    mesh = plsc.VectorSubcoreMesh(
        core_axis_name="c", subcore_axis_name="s",
        num_cores=_NC, num_subcores=_NS,
    )

    @functools.partial(
        pl.kernel,
        out_type=jax.ShapeDtypeStruct((n,), jnp.float32),
        mesh=mesh,
        scratch_types=[
            pltpu.VMEM((_TBL_PAD,), jnp.float32),
            [pltpu.VMEM((per_w // _NB,), jnp.int32) for _ in range(2)],
            [pltpu.VMEM((per_w // _NB,), jnp.float32) for _ in range(2)],
            [pltpu.SemaphoreType.DMA for _ in range(2)],
            [pltpu.SemaphoreType.DMA for _ in range(2)],
        ],
        compiler_params=pltpu.CompilerParams(
            needs_layout_passes=False,
            disable_bounds_checks=True,
            disable_semaphore_checks=True,
            skip_device_barrier=True,
        ),
    )
    def lookup(an_hbm, tbl_hbm, out_hbm, tbl_v, idx_v, val_v, in_sem, out_sem):
        wid = lax.axis_index("s") * _NC + lax.axis_index("c")
        base = wid * per_w
        chunk = per_w // _NB

        in_cp = [None] * _NB
        out_cp = [None] * _NB
        in_cp[0] = pltpu.async_copy(
            an_hbm.at[pl.ds(base, chunk)], idx_v[0], in_sem[0]
        )

        pltpu.sync_copy(tbl_hbm, tbl_v.at[pl.ds(0, tbl_n)])
        for j in range(_TBL_PAD // _L):
            sl = pl.ds(j * _L, _L)
            tbl_v[sl] = tbl_v[sl] * _NORM

        for k in range(_NB):
            b = k % 2
            if k + 1 < _NB:
                nb = (k + 1) % 2
                in_cp[k + 1] = pltpu.async_copy(
                    an_hbm.at[pl.ds(base + (k + 1) * chunk, chunk)],
                    idx_v[nb], in_sem[nb],
                )
            in_cp[k].wait()
            if k >= 2:
                out_cp[k - 2].wait()

            @plsc.parallel_loop(0, chunk, step=_L, unroll=32)
            def _(i, ib=idx_v[b], vb=val_v[b]):
                sl = pl.ds(i, _L)
                vb[sl] = plsc.load_gather(tbl_v, [ib[sl]])

            out_cp[k] = pltpu.async_copy(
                val_v[b], out_hbm.at[pl.ds(base + k * chunk, chunk)],
                out_sem[b],
            )

        out_cp[_NB - 2].wait()
        out_cp[_NB - 1].wait()

    return lookup


def kernel(atomic_numbers, alpha_table):
    return _make_lookup(atomic_numbers.shape[0], alpha_table.shape[0])(
        atomic_numbers, alpha_table
    )
